# Initial kernel scaffold; baseline (speedup 1.0000x reference)
import jax
import jax.numpy as jnp
from jax.experimental import pallas as pl


def kernel(x, edge_index, W_msg, W_self, b):
    raise NotImplementedError("write your pallas kernel here")


# speedup vs baseline: 5.6431x; 5.6431x over previous
import functools

import jax
import jax.numpy as jnp
from jax import lax
from jax.experimental import pallas as pl
from jax.experimental.pallas import tpu as pltpu
from jax.experimental.pallas import tpu_sc as plsc

_N = 10000
_D = 128
_E = 320000
_NC = 2
_NS = 16
_NW = _NC * _NS
_CHUNK = 128
_CHUNKS_PER_TILE = -(-_E // (_NW * _CHUNK))
_EDGES_PER_TILE = _CHUNKS_PER_TILE * _CHUNK
_E_PAD = _NW * _EDGES_PER_TILE
_AGG_ROWS = 10112
_ZBLOCKS = _AGG_ROWS // _CHUNK
_ROWS_PER_TILE = _N // _NS
_RCHUNK = 125

def _sc_body(x_hbm, src_hbm, dst_hbm, zeros_hbm, out_hbm,
             src_v, dst_v, rows_v, agg_sh, sem):
    c = lax.axis_index("c")
    s = lax.axis_index("s")
    wid = c * _NS + s

    pltpu.sync_copy(zeros_hbm, rows_v)
    for k in range(-(-_ZBLOCKS // _NS)):
        blk = s + k * _NS

        @pl.when(blk < _ZBLOCKS)
        def _():
            pltpu.sync_copy(rows_v, agg_sh.at[pl.ds(blk * _CHUNK, _CHUNK)])

    plsc.subcore_barrier()

    pltpu.sync_copy(src_hbm.at[wid], src_v)
    pltpu.sync_copy(dst_hbm.at[wid], dst_v)

    def body(j, carry):
        pltpu.async_copy(x_hbm.at[src_v.at[j]], rows_v, sem).wait()
        pltpu.sync_copy(rows_v, agg_sh.at[dst_v.at[j]], add=True)
        return carry

    lax.fori_loop(0, _CHUNKS_PER_TILE, body, 0, unroll=False)

    plsc.subcore_barrier()

    nfull = _N // _CHUNK
    tail = _N - nfull * _CHUNK
    for k in range(-(-(nfull + 1) // _NS)):
        blk = s + k * _NS
        r0 = pl.multiple_of(blk * _CHUNK, _CHUNK)
        o0 = pl.multiple_of(c * _N + r0, 16)

        @pl.when(blk < nfull)
        def _():
            pltpu.sync_copy(agg_sh.at[pl.ds(r0, _CHUNK)], rows_v)
            pltpu.sync_copy(rows_v, out_hbm.at[pl.ds(o0, _CHUNK)])

        @pl.when(blk == nfull)
        def _():
            pltpu.sync_copy(agg_sh.at[pl.ds(r0, tail)], rows_v.at[pl.ds(0, tail)])
            pltpu.sync_copy(rows_v.at[pl.ds(0, tail)],
                            out_hbm.at[pl.ds(o0, tail)])


@functools.cache
def _sc_segment_sum():
    mesh = plsc.VectorSubcoreMesh(
        core_axis_name="c", subcore_axis_name="s", num_cores=_NC, num_subcores=_NS
    )
    return pl.kernel(
        _sc_body,
        out_type=jax.ShapeDtypeStruct((_NC * _N, _D), jnp.float32),
        mesh=mesh,
        scratch_types=[
            pltpu.VMEM((_CHUNKS_PER_TILE, _CHUNK), jnp.int32),
            pltpu.VMEM((_CHUNKS_PER_TILE, _CHUNK), jnp.int32),
            pltpu.VMEM((_CHUNK, _D), jnp.float32),
            pltpu.VMEM_SHARED((_AGG_ROWS, _D), jnp.float32),
            pltpu.SemaphoreType.DMA,
        ],
    )


_TC_ROWS = 1000


def _tc_body(x_ref, agg_ref, wm_ref, ws_ref, b_ref, o_ref):
    agg = agg_ref[0] + agg_ref[1]
    acc = jnp.dot(x_ref[...], ws_ref[...], preferred_element_type=jnp.float32)
    acc = acc + jnp.dot(agg, wm_ref[...], preferred_element_type=jnp.float32)
    o_ref[...] = jnp.maximum(acc + b_ref[...], 0.0)


@jax.jit
def _tc_combine(x, agg2, W_msg, W_self, b2):
    return pl.pallas_call(
        _tc_body,
        grid=(_N // _TC_ROWS,),
        in_specs=[
            pl.BlockSpec((_TC_ROWS, _D), lambda i: (i, 0)),
            pl.BlockSpec((_NC, _TC_ROWS, _D), lambda i: (0, i, 0)),
            pl.BlockSpec((_D, _D), lambda i: (0, 0)),
            pl.BlockSpec((_D, _D), lambda i: (0, 0)),
            pl.BlockSpec((1, _D), lambda i: (0, 0)),
        ],
        out_specs=pl.BlockSpec((_TC_ROWS, _D), lambda i: (i, 0)),
        out_shape=jax.ShapeDtypeStruct((_N, _D), jnp.float32),
    )(x, agg2, W_msg, W_self, b2)


def kernel(x, edge_index, W_msg, W_self, b):
    src = edge_index[0].astype(jnp.int32)
    dst = edge_index[1].astype(jnp.int32)
    pad = _E_PAD - _E
    src = jnp.concatenate([src, jnp.zeros((pad,), jnp.int32)])
    dst = jnp.concatenate([dst, jnp.full((pad,), _N, jnp.int32)])
    src = src.reshape(_NW, _CHUNKS_PER_TILE, _CHUNK)
    dst = dst.reshape(_NW, _CHUNKS_PER_TILE, _CHUNK)
    zeros_blk = jnp.zeros((_CHUNK, _D), jnp.float32)
    agg2 = _sc_segment_sum()(x, src, dst, zeros_blk).reshape(_NC, _N, _D)
    return _tc_combine(x, agg2, W_msg, W_self, b.reshape(1, _D))
